# baseline (speedup 1.0000x reference)
import jax
import jax.numpy as jnp
from jax import lax
from jax.experimental import pallas as pl
from jax.experimental.pallas import tpu as pltpu
from jax.experimental.pallas import tpu_sc as plsc

L_SEQ, B_SZ = 200, 1024
NC, NS = 2, 16
NW = NC * NS
CHUNK = 128
NBB = B_SZ // CHUNK
NCH = L_SEQ * NBB // NW
WIDTHS = (64, 64, 32, 32, 32, 32, 32)
COLS = (0, 64, 128, 160, 192, 224, 256)
OUT_D = 288
NT = 7
NFT = OUT_D // 8


def _body(*refs):
    tables = refs[0:NT]
    idx_hbm = refs[NT:2 * NT]
    out = refs[2 * NT]
    p = 2 * NT + 1
    idxb = (refs[p:p + NT], refs[p + NT:p + 2 * NT])
    p += 2 * NT
    rows = (refs[p:p + NT], refs[p + NT:p + 2 * NT])
    p += 2 * NT
    tbufs = refs[p:p + NT]
    p += NT
    sem_g = (refs[p], refs[p + 1])
    sem_w = refs[p + 2]

    wid = lax.axis_index("s") * NC + lax.axis_index("c")
    l0 = (wid // NBB) * NCH
    bt = wid % NBB
    b0 = bt * CHUNK

    lane = jax.lax.broadcasted_iota(jnp.int32, (16,), 0)
    toks = [lane + (g * 16) for g in range(8)]

    def load_idx(chunk, par):
        for t in range(NT):
            pltpu.sync_copy(idx_hbm[t].at[l0 + chunk, pl.ds(b0, CHUNK)],
                            idxb[par][t])

    def fire_gathers(par):
        for t in range(NT):
            pltpu.async_copy(tables[t].at[idxb[par][t]], rows[par][t],
                             sem_g[par])

    def wait_gathers(par):
        for t in range(NT):
            pltpu.make_async_copy(tables[t].at[idxb[par][t]], rows[par][t],
                                  sem_g[par]).wait()

    def transpose_t(t, par):
        src = rows[par][t]
        dst = tbufs[t]

        def f4_body(i, fvec):
            for u in range(4):
                fv = fvec + u
                f = i * 4 + u
                ft = f // 8
                fs = f % 8
                for g in range(8):
                    v = plsc.load_gather(src, [toks[g], fv])
                    dst[ft, fs, pl.ds(g * 16, 16)] = v
            return fvec + 4

        lax.fori_loop(0, WIDTHS[t] // 4, f4_body, jnp.zeros((16,), jnp.int32))

    def fire_writes(chunk):
        for t in range(NT):
            pltpu.async_copy(
                tbufs[t],
                out.at[l0 + chunk, pl.ds(COLS[t] // 8, WIDTHS[t] // 8), bt],
                sem_w)

    def wait_writes():
        for t in range(NT):
            pltpu.make_async_copy(
                tbufs[t],
                out.at[l0, pl.ds(COLS[t] // 8, WIDTHS[t] // 8), bt],
                sem_w).wait()

    load_idx(0, 0)
    fire_gathers(0)
    load_idx(1, 1)

    n_iter = NCH // 2

    def process(chunk, par, j, last_fire):
        wait_gathers(par)

        @pl.when(last_fire)
        def _():
            load_idx(chunk + 2, par)
            fire_gathers(par)

        @pl.when(j > 0)
        def _():
            wait_writes()

        for t in range(NT):
            transpose_t(t, par)
        fire_writes(chunk)

    def body(j, carry):
        a = 2 * j
        wait_gathers(0)
        fire_gathers(1)

        @pl.when(j > 0)
        def _():
            wait_writes()

        for t in range(NT):
            transpose_t(t, 0)
        fire_writes(a)

        wait_gathers(1)

        @pl.when(j < n_iter - 1)
        def _():
            load_idx(a + 2, 0)
            fire_gathers(0)
            load_idx(a + 3, 1)

        wait_writes()
        for t in range(NT):
            transpose_t(t, 1)
        fire_writes(a + 1)
        return carry

    lax.fori_loop(0, n_iter, body, 0)
    wait_writes()


def kernel(attribute_key, attribute_word, attribute_word_local_fw_pos,
           attribute_word_local_bw_pos, attribute_kv_pos, attribute_kw_pos,
           attribute_word_tag, field_key_table, field_word_table,
           local_pos_fw_table, local_pos_bw_table, kv_pos_table,
           kw_pos_table, field_tag_table):
    tables = (field_word_table, field_key_table, local_pos_fw_table,
              local_pos_bw_table, kv_pos_table, kw_pos_table, field_tag_table)
    idxs = (attribute_word, attribute_key, attribute_word_local_fw_pos,
            attribute_word_local_bw_pos, attribute_kv_pos,
            attribute_kw_pos, attribute_word_tag)

    mesh = plsc.VectorSubcoreMesh(core_axis_name="c", subcore_axis_name="s")
    scratch = (
        [pltpu.VMEM((CHUNK,), jnp.int32) for _ in range(2 * NT)]
        + [pltpu.VMEM((CHUNK, w), jnp.float32) for w in WIDTHS]
        + [pltpu.VMEM((CHUNK, w), jnp.float32) for w in WIDTHS]
        + [pltpu.VMEM((w // 8, 8, 128), jnp.float32) for w in WIDTHS]
        + [pltpu.SemaphoreType.DMA for _ in range(3)]
    )
    out5 = pl.kernel(
        _body,
        out_type=jax.ShapeDtypeStruct((L_SEQ, NFT, NBB, 8, 128), jnp.float32),
        mesh=mesh,
        scratch_types=scratch,
        compiler_params=pltpu.CompilerParams(use_tc_tiling_on_sc=False,
                                             needs_layout_passes=False),
    )(*tables, *idxs)
    return out5.transpose(0, 2, 4, 1, 3).reshape(L_SEQ, B_SZ, OUT_D)


# speedup vs baseline: 1.7139x; 1.7139x over previous
import jax
import jax.numpy as jnp
from jax import lax
from jax.experimental import pallas as pl
from jax.experimental.pallas import tpu as pltpu
from jax.experimental.pallas import tpu_sc as plsc

L_SEQ, B_SZ = 200, 1024
NC, NS = 2, 16
NW = NC * NS
CHUNK = 128
NBB = B_SZ // CHUNK
NCH = L_SEQ * NBB // NW
WIDTHS = (64, 64, 32, 32, 32, 32, 32)
COLS = (0, 64, 128, 160, 192, 224, 256)
OUT_D = 288
NT = 7
NFT = OUT_D // 8


def _body(*refs):
    tables = refs[0:NT]
    idx_hbm = refs[NT:2 * NT]
    out = refs[2 * NT]
    p = 2 * NT + 1
    idxb = (refs[p:p + NT], refs[p + NT:p + 2 * NT])
    p += 2 * NT
    rows = (refs[p:p + NT], refs[p + NT:p + 2 * NT])
    p += 2 * NT
    tbufs = refs[p:p + NT]
    p += NT
    sem_g = (refs[p], refs[p + 1])
    sem_w = refs[p + 2]

    wid = lax.axis_index("s") * NC + lax.axis_index("c")
    l0 = (wid // NBB) * NCH
    bt = wid % NBB
    b0 = bt * CHUNK

    lane = jax.lax.broadcasted_iota(jnp.int32, (16,), 0)
    ftv = [(c * 16 + lane) // 8 for c in range(4)]
    fsv = [(c * 16 + lane) % 8 for c in range(4)]

    def load_idx(chunk, par):
        for t in range(NT):
            pltpu.sync_copy(idx_hbm[t].at[l0 + chunk, pl.ds(b0, CHUNK)],
                            idxb[par][t])

    def fire_gathers(par):
        for t in range(NT):
            pltpu.async_copy(tables[t].at[idxb[par][t]], rows[par][t],
                             sem_g[par])

    def wait_gathers(par):
        for t in range(NT):
            pltpu.make_async_copy(tables[t].at[idxb[par][t]], rows[par][t],
                                  sem_g[par]).wait()

    def transpose_t(t, par):
        src = rows[par][t]
        dst = tbufs[t]
        ncg = WIDTHS[t] // 16

        def tok_body(i, tokvec):
            for u in range(2):
                tok = i * 2 + u
                tv = tokvec + u
                for c in range(ncg):
                    v = src[tok, pl.ds(c * 16, 16)]
                    plsc.store_scatter(dst, [ftv[c], fsv[c], tv], v)
            return tokvec + 2

        lax.fori_loop(0, CHUNK // 2, tok_body, jnp.zeros((16,), jnp.int32))

    def fire_writes(chunk):
        for t in range(NT):
            pltpu.async_copy(
                tbufs[t].at[slice(None), slice(None), pl.ds(0, 128)],
                out.at[l0 + chunk, pl.ds(COLS[t] // 8, WIDTHS[t] // 8), bt],
                sem_w)

    def wait_writes():
        for t in range(NT):
            pltpu.make_async_copy(
                tbufs[t].at[slice(None), slice(None), pl.ds(0, 128)],
                out.at[l0, pl.ds(COLS[t] // 8, WIDTHS[t] // 8), bt],
                sem_w).wait()

    load_idx(0, 0)
    fire_gathers(0)
    load_idx(1, 1)

    n_iter = NCH // 2


    def body(j, carry):
        a = 2 * j
        wait_gathers(0)
        fire_gathers(1)

        @pl.when(j > 0)
        def _():
            wait_writes()

        for t in range(NT):
            transpose_t(t, 0)
        fire_writes(a)

        wait_gathers(1)

        @pl.when(j < n_iter - 1)
        def _():
            load_idx(a + 2, 0)
            fire_gathers(0)
            load_idx(a + 3, 1)

        wait_writes()
        for t in range(NT):
            transpose_t(t, 1)
        fire_writes(a + 1)
        return carry

    lax.fori_loop(0, n_iter, body, 0)
    wait_writes()


def kernel(attribute_key, attribute_word, attribute_word_local_fw_pos,
           attribute_word_local_bw_pos, attribute_kv_pos, attribute_kw_pos,
           attribute_word_tag, field_key_table, field_word_table,
           local_pos_fw_table, local_pos_bw_table, kv_pos_table,
           kw_pos_table, field_tag_table):
    tables = (field_word_table, field_key_table, local_pos_fw_table,
              local_pos_bw_table, kv_pos_table, kw_pos_table, field_tag_table)
    idxs = (attribute_word, attribute_key, attribute_word_local_fw_pos,
            attribute_word_local_bw_pos, attribute_kv_pos,
            attribute_kw_pos, attribute_word_tag)

    mesh = plsc.VectorSubcoreMesh(core_axis_name="c", subcore_axis_name="s")
    scratch = (
        [pltpu.VMEM((CHUNK,), jnp.int32) for _ in range(2 * NT)]
        + [pltpu.VMEM((CHUNK, w), jnp.float32) for w in WIDTHS]
        + [pltpu.VMEM((CHUNK, w), jnp.float32) for w in WIDTHS]
        + [pltpu.VMEM((w // 8, 8, 129), jnp.float32) for w in WIDTHS]
        + [pltpu.SemaphoreType.DMA for _ in range(3)]
    )
    out5 = pl.kernel(
        _body,
        out_type=jax.ShapeDtypeStruct((L_SEQ, NFT, NBB, 8, 128), jnp.float32),
        mesh=mesh,
        scratch_types=scratch,
        compiler_params=pltpu.CompilerParams(use_tc_tiling_on_sc=False,
                                             needs_layout_passes=False),
    )(*tables, *idxs)
    return out5.transpose(0, 2, 4, 1, 3).reshape(L_SEQ, B_SZ, OUT_D)


# speedup vs baseline: 1.7996x; 1.0500x over previous
import jax
import jax.numpy as jnp
from jax import lax
from jax.experimental import pallas as pl
from jax.experimental.pallas import tpu as pltpu
from jax.experimental.pallas import tpu_sc as plsc

L_SEQ, B_SZ = 200, 1024
NC, NS = 2, 16
NW = NC * NS
CHUNK = 128
NBB = B_SZ // CHUNK
NCH = L_SEQ * NBB // NW
WIDTHS = (64, 64, 32, 32, 32, 32, 32)
COLS = (0, 64, 128, 160, 192, 224, 256)
OUT_D = 288
NT = 7
NFT = OUT_D // 8


def _body(*refs):
    tables = refs[0:NT]
    idx_hbm = refs[NT:2 * NT]
    out = refs[2 * NT]
    p = 2 * NT + 1
    idxb = (refs[p:p + NT], refs[p + NT:p + 2 * NT])
    p += 2 * NT
    rows = (refs[p:p + NT], refs[p + NT:p + 2 * NT])
    p += 2 * NT
    tbufs = refs[p:p + NT]
    p += NT
    sem_g = (refs[p], refs[p + 1])
    sem_w = refs[p + 2]

    wid = lax.axis_index("s") * NC + lax.axis_index("c")
    l0 = (wid // NBB) * NCH
    bt = wid % NBB
    b0 = bt * CHUNK

    lane = jax.lax.broadcasted_iota(jnp.int32, (16,), 0)
    ftv = [(c * 16 + lane) // 8 for c in range(4)]
    fsv = [(c * 16 + lane) % 8 for c in range(4)]

    def load_idx(chunk, par):
        for t in range(NT):
            pltpu.sync_copy(idx_hbm[t].at[l0 + chunk, pl.ds(b0, CHUNK)],
                            idxb[par][t])

    def fire_gathers(par):
        for t in range(NT):
            pltpu.async_copy(tables[t].at[idxb[par][t]], rows[par][t],
                             sem_g[par])

    def wait_gathers(par):
        for t in range(NT):
            pltpu.make_async_copy(tables[t].at[idxb[par][t]], rows[par][t],
                                  sem_g[par]).wait()

    def transpose_t(t, par):
        src = rows[par][t]
        dst = tbufs[t]
        ncg = WIDTHS[t] // 16

        def tok_body(i, tokvec):
            for u in range(2):
                tok = i * 2 + u
                tv = tokvec + u
                for c in range(ncg):
                    v = src[tok, pl.ds(c * 16, 16)]
                    plsc.store_scatter(dst, [ftv[c], fsv[c], tv], v)
            return tokvec + 2

        lax.fori_loop(0, CHUNK // 2, tok_body, jnp.zeros((16,), jnp.int32))

    def fire_writes(chunk):
        for t in range(NT):
            pltpu.async_copy(
                tbufs[t].at[slice(None), slice(None), pl.ds(0, 128)],
                out.at[l0 + chunk, pl.ds(COLS[t] // 8, WIDTHS[t] // 8), bt],
                sem_w)

    def wait_writes():
        for t in range(NT):
            pltpu.make_async_copy(
                tbufs[t].at[slice(None), slice(None), pl.ds(0, 128)],
                out.at[l0, pl.ds(COLS[t] // 8, WIDTHS[t] // 8), bt],
                sem_w).wait()

    load_idx(0, 0)
    fire_gathers(0)
    load_idx(1, 1)

    n_iter = NCH // 2


    def body(j, carry):
        a = 2 * j
        wait_gathers(0)
        fire_gathers(1)

        @pl.when(j > 0)
        def _():
            wait_writes()

        for t in range(NT):
            transpose_t(t, 0)
        fire_writes(a)

        wait_gathers(1)

        @pl.when(j < n_iter - 1)
        def _():
            load_idx(a + 2, 0)
            fire_gathers(0)
            load_idx(a + 3, 1)

        wait_writes()
        for t in range(NT):
            transpose_t(t, 1)
        fire_writes(a + 1)
        return carry

    lax.fori_loop(0, n_iter, body, 0)
    wait_writes()


def kernel(attribute_key, attribute_word, attribute_word_local_fw_pos,
           attribute_word_local_bw_pos, attribute_kv_pos, attribute_kw_pos,
           attribute_word_tag, field_key_table, field_word_table,
           local_pos_fw_table, local_pos_bw_table, kv_pos_table,
           kw_pos_table, field_tag_table):
    field_word_table = jnp.pad(field_word_table, ((0, 0), (0, 64)))
    tables = (field_word_table, field_key_table, local_pos_fw_table,
              local_pos_bw_table, kv_pos_table, kw_pos_table, field_tag_table)
    idxs = (attribute_word, attribute_key, attribute_word_local_fw_pos,
            attribute_word_local_bw_pos, attribute_kv_pos,
            attribute_kw_pos, attribute_word_tag)

    mesh = plsc.VectorSubcoreMesh(core_axis_name="c", subcore_axis_name="s")
    scratch = (
        [pltpu.VMEM((CHUNK,), jnp.int32) for _ in range(2 * NT)]
        + [pltpu.VMEM((CHUNK, 128 if i == 0 else w), jnp.float32)
           for i, w in enumerate(WIDTHS)]
        + [pltpu.VMEM((CHUNK, 128 if i == 0 else w), jnp.float32)
           for i, w in enumerate(WIDTHS)]
        + [pltpu.VMEM((w // 8, 8, 129), jnp.float32) for w in WIDTHS]
        + [pltpu.SemaphoreType.DMA for _ in range(3)]
    )
    out5 = pl.kernel(
        _body,
        out_type=jax.ShapeDtypeStruct((L_SEQ, NFT, NBB, 8, 128), jnp.float32),
        mesh=mesh,
        scratch_types=scratch,
        compiler_params=pltpu.CompilerParams(use_tc_tiling_on_sc=False,
                                             needs_layout_passes=False),
    )(*tables, *idxs)
    return out5.transpose(0, 2, 4, 1, 3).reshape(L_SEQ, B_SZ, OUT_D)
